# baseline (speedup 1.0000x reference)
import functools

import jax
import jax.numpy as jnp
from jax.experimental import pallas as pl
from jax.experimental.pallas import tpu as pltpu

T = 2048
D = 1024
E = 8
K = 2
F = 704
NEXP = 10

TT = 512


def _moe_body(x_ref, wr_ref, wg_ref, wu_ref, wd_ref, out_ref, cw_ref):
    e = pl.program_id(1)

    @pl.when(e == 0)
    def _router():
        xb = x_ref[...]
        logits = jnp.dot(xb, wr_ref[...], preferred_element_type=jnp.float32)
        lane = jax.lax.broadcasted_iota(jnp.int32, (TT, 128), 1)
        mask = lane < E
        lm = jnp.where(mask, logits, -1e30)
        mx = jnp.max(lm, axis=1, keepdims=True)
        p = jnp.where(mask, jnp.exp(lm - mx), 0.0)
        sc = p / jnp.sum(p, axis=1, keepdims=True)
        a1 = jnp.argmax(sc, axis=1)
        oh1 = lane == a1[:, None]
        m1 = jnp.sum(jnp.where(oh1, sc, 0.0), axis=1, keepdims=True)
        sc2 = jnp.where(oh1, -1.0, sc)
        a2 = jnp.argmax(sc2, axis=1)
        oh2 = lane == a2[:, None]
        m2 = jnp.sum(jnp.where(oh2, sc, 0.0), axis=1, keepdims=True)
        wsum = m1 + m2
        cw_ref[...] = (jnp.where(oh1, m1, 0.0) + jnp.where(oh2, m2, 0.0)) / wsum

    xb = x_ref[...]
    hg = jnp.dot(xb, wg_ref[0], preferred_element_type=jnp.float32)
    hu = jnp.dot(xb, wu_ref[0], preferred_element_type=jnp.float32)
    lane = jax.lax.broadcasted_iota(jnp.int32, (TT, 128), 1)
    wcol = jnp.sum(jnp.where(lane == e, cw_ref[...], 0.0), axis=1, keepdims=True)
    wcol = jnp.where(e < E, wcol, 1.0)
    h = (hg * jax.nn.sigmoid(hg)) * hu * wcol
    contrib = jnp.dot(h, wd_ref[0], preferred_element_type=jnp.float32)

    @pl.when(e == 0)
    def _init():
        out_ref[...] = contrib

    @pl.when(e > 0)
    def _acc():
        out_ref[...] = out_ref[...] + contrib


def kernel(hidden_states, W_router, Wg, Wu, Wd, Ws_g, Ws_u, Ws_d):
    x = hidden_states
    wr_pad = jnp.zeros((D, 128), jnp.float32).at[:, :E].set(W_router)
    wg_all = jnp.concatenate(
        [Wg, Ws_g.reshape(D, 2, F).transpose(1, 0, 2)], axis=0)
    wu_all = jnp.concatenate(
        [Wu, Ws_u.reshape(D, 2, F).transpose(1, 0, 2)], axis=0)
    wd_all = jnp.concatenate([Wd, Ws_d.reshape(2, F, D)], axis=0)

    grid = (T // TT, NEXP)
    out = pl.pallas_call(
        _moe_body,
        grid=grid,
        in_specs=[
            pl.BlockSpec((TT, D), lambda t, e: (t, 0)),
            pl.BlockSpec((D, 128), lambda t, e: (0, 0)),
            pl.BlockSpec((1, D, F), lambda t, e: (e, 0, 0)),
            pl.BlockSpec((1, D, F), lambda t, e: (e, 0, 0)),
            pl.BlockSpec((1, F, D), lambda t, e: (e, 0, 0)),
        ],
        out_specs=pl.BlockSpec((TT, D), lambda t, e: (t, 0)),
        out_shape=jax.ShapeDtypeStruct((T, D), jnp.float32),
        scratch_shapes=[pltpu.VMEM((TT, 128), jnp.float32)],
        compiler_params=pltpu.CompilerParams(
            dimension_semantics=("parallel", "arbitrary")),
    )(x, wr_pad, wg_all, wu_all, wd_all)
    return out


# speedup vs baseline: 1.1837x; 1.1837x over previous
import functools

import jax
import jax.numpy as jnp
from jax.experimental import pallas as pl
from jax.experimental.pallas import tpu as pltpu

T = 2048
D = 1024
E = 8
K = 2
F = 704
NEXP = 10

TT = 512


def _moe_body(x_ref, wr_ref, wg_ref, wu_ref, wd_ref, out_ref, cw_ref):
    e = pl.program_id(1)

    @pl.when(e == 0)
    def _router():
        xb = x_ref[...]
        logits = jnp.dot(xb, wr_ref[...], preferred_element_type=jnp.float32)
        lane = jax.lax.broadcasted_iota(jnp.int32, (TT, 128), 1)
        mask = lane < E
        lm = jnp.where(mask, logits, -1e30)
        mx = jnp.max(lm, axis=1, keepdims=True)
        p = jnp.where(mask, jnp.exp(lm - mx), 0.0)
        sc = p / jnp.sum(p, axis=1, keepdims=True)
        a1 = jnp.argmax(sc, axis=1)
        oh1 = lane == a1[:, None]
        m1 = jnp.sum(jnp.where(oh1, sc, 0.0), axis=1, keepdims=True)
        sc2 = jnp.where(oh1, -1.0, sc)
        a2 = jnp.argmax(sc2, axis=1)
        oh2 = lane == a2[:, None]
        m2 = jnp.sum(jnp.where(oh2, sc, 0.0), axis=1, keepdims=True)
        wsum = m1 + m2
        cw_ref[...] = (jnp.where(oh1, m1, 0.0) + jnp.where(oh2, m2, 0.0)) / wsum

    xb = x_ref[...].astype(jnp.bfloat16)
    hg = jnp.dot(xb, wg_ref[0], preferred_element_type=jnp.float32)
    hu = jnp.dot(xb, wu_ref[0], preferred_element_type=jnp.float32)
    lane = jax.lax.broadcasted_iota(jnp.int32, (TT, 128), 1)
    wcol = jnp.sum(jnp.where(lane == e, cw_ref[...], 0.0), axis=1, keepdims=True)
    wcol = jnp.where(e < E, wcol, 1.0)
    h = ((hg * jax.nn.sigmoid(hg)) * hu * wcol).astype(jnp.bfloat16)
    contrib = jnp.dot(h, wd_ref[0], preferred_element_type=jnp.float32)

    @pl.when(e == 0)
    def _init():
        out_ref[...] = contrib

    @pl.when(e > 0)
    def _acc():
        out_ref[...] = out_ref[...] + contrib


def kernel(hidden_states, W_router, Wg, Wu, Wd, Ws_g, Ws_u, Ws_d):
    x = hidden_states
    wr_pad = jnp.zeros((D, 128), jnp.float32).at[:, :E].set(W_router)
    wg_all = jnp.concatenate(
        [Wg, Ws_g.reshape(D, 2, F).transpose(1, 0, 2)], axis=0).astype(jnp.bfloat16)
    wu_all = jnp.concatenate(
        [Wu, Ws_u.reshape(D, 2, F).transpose(1, 0, 2)], axis=0).astype(jnp.bfloat16)
    wd_all = jnp.concatenate([Wd, Ws_d.reshape(2, F, D)], axis=0).astype(jnp.bfloat16)

    grid = (T // TT, NEXP)
    out = pl.pallas_call(
        _moe_body,
        grid=grid,
        in_specs=[
            pl.BlockSpec((TT, D), lambda t, e: (t, 0)),
            pl.BlockSpec((D, 128), lambda t, e: (0, 0)),
            pl.BlockSpec((1, D, F), lambda t, e: (e, 0, 0)),
            pl.BlockSpec((1, D, F), lambda t, e: (e, 0, 0)),
            pl.BlockSpec((1, F, D), lambda t, e: (e, 0, 0)),
        ],
        out_specs=pl.BlockSpec((TT, D), lambda t, e: (t, 0)),
        out_shape=jax.ShapeDtypeStruct((T, D), jnp.float32),
        scratch_shapes=[pltpu.VMEM((TT, 128), jnp.float32)],
        compiler_params=pltpu.CompilerParams(
            dimension_semantics=("parallel", "arbitrary")),
    )(x, wr_pad, wg_all, wu_all, wd_all)
    return out


# speedup vs baseline: 1.6014x; 1.3529x over previous
import jax
import jax.numpy as jnp
from jax.experimental import pallas as pl
from jax.experimental.pallas import tpu as pltpu

T = 2048
D = 1024
E = 8
F = 704


def _routed_body(x_ref, wr_ref, wg_ref, wu_ref, wd_ref, out_ref, cw_ref):
    e = pl.program_id(0)

    @pl.when(e == 0)
    def _router():
        xb = x_ref[...]
        logits = jnp.dot(xb, wr_ref[...], preferred_element_type=jnp.float32)
        lane = jax.lax.broadcasted_iota(jnp.int32, (T, 128), 1)
        mask = lane < E
        lm = jnp.where(mask, logits, -1e30)
        mx = jnp.max(lm, axis=1, keepdims=True)
        p = jnp.where(mask, jnp.exp(lm - mx), 0.0)
        sc = p / jnp.sum(p, axis=1, keepdims=True)
        a1 = jnp.argmax(sc, axis=1)
        oh1 = lane == a1[:, None]
        m1 = jnp.sum(jnp.where(oh1, sc, 0.0), axis=1, keepdims=True)
        sc2 = jnp.where(oh1, -1.0, sc)
        a2 = jnp.argmax(sc2, axis=1)
        oh2 = lane == a2[:, None]
        m2 = jnp.sum(jnp.where(oh2, sc, 0.0), axis=1, keepdims=True)
        wsum = m1 + m2
        cw_ref[...] = (jnp.where(oh1, m1, 0.0) + jnp.where(oh2, m2, 0.0)) / wsum

    xb = x_ref[...]
    hg = jnp.dot(xb, wg_ref[0], preferred_element_type=jnp.float32)
    hu = jnp.dot(xb, wu_ref[0], preferred_element_type=jnp.float32)
    lane = jax.lax.broadcasted_iota(jnp.int32, (T, 128), 1)
    wcol = jnp.sum(jnp.where(lane == e, cw_ref[...], 0.0), axis=1, keepdims=True)
    h = (hg * jax.nn.sigmoid(hg)) * hu * wcol
    contrib = jnp.dot(h, wd_ref[0], preferred_element_type=jnp.float32)

    @pl.when(e == 0)
    def _init():
        out_ref[...] = contrib

    @pl.when(e > 0)
    def _acc():
        out_ref[...] = out_ref[...] + contrib


def _shared_body(x_ref, wsg_ref, wsu_ref, wsd_ref, routed_ref, out_ref):
    xb = x_ref[...]
    hg = jnp.dot(xb, wsg_ref[...], preferred_element_type=jnp.float32)
    hu = jnp.dot(xb, wsu_ref[...], preferred_element_type=jnp.float32)
    h = (hg * jax.nn.sigmoid(hg)) * hu
    contrib = jnp.dot(h, wsd_ref[...], preferred_element_type=jnp.float32)
    out_ref[...] = routed_ref[...] + contrib


def kernel(hidden_states, W_router, Wg, Wu, Wd, Ws_g, Ws_u, Ws_d):
    x = hidden_states
    wr_pad = jnp.zeros((D, 128), jnp.float32).at[:, :E].set(W_router)

    routed = pl.pallas_call(
        _routed_body,
        grid=(E,),
        in_specs=[
            pl.BlockSpec((T, D), lambda e: (0, 0)),
            pl.BlockSpec((D, 128), lambda e: (0, 0)),
            pl.BlockSpec((1, D, F), lambda e: (e, 0, 0)),
            pl.BlockSpec((1, D, F), lambda e: (e, 0, 0)),
            pl.BlockSpec((1, F, D), lambda e: (e, 0, 0)),
        ],
        out_specs=pl.BlockSpec((T, D), lambda e: (0, 0)),
        out_shape=jax.ShapeDtypeStruct((T, D), jnp.float32),
        scratch_shapes=[pltpu.VMEM((T, 128), jnp.float32)],
        compiler_params=pltpu.CompilerParams(
            dimension_semantics=("arbitrary",)),
    )(x, wr_pad, Wg, Wu, Wd)

    FS = Ws_g.shape[1]
    ST = 512
    out = pl.pallas_call(
        _shared_body,
        grid=(T // ST,),
        in_specs=[
            pl.BlockSpec((ST, D), lambda t: (t, 0)),
            pl.BlockSpec((D, FS), lambda t: (0, 0)),
            pl.BlockSpec((D, FS), lambda t: (0, 0)),
            pl.BlockSpec((FS, D), lambda t: (0, 0)),
            pl.BlockSpec((ST, D), lambda t: (t, 0)),
        ],
        out_specs=pl.BlockSpec((ST, D), lambda t: (t, 0)),
        out_shape=jax.ShapeDtypeStruct((T, D), jnp.float32),
        compiler_params=pltpu.CompilerParams(
            dimension_semantics=("arbitrary",)),
    )(x, Ws_g, Ws_u, Ws_d, routed)
    return out
